# Initial kernel scaffold; baseline (speedup 1.0000x reference)
import jax
import jax.numpy as jnp
from jax.experimental import pallas as pl


def kernel(idx, targets, table):
    raise NotImplementedError("write your pallas kernel here")


# speedup vs baseline: 1.3656x; 1.3656x over previous
import functools

import jax
import jax.numpy as jnp
from jax import lax
from jax.experimental import pallas as pl
from jax.experimental.pallas import tpu as pltpu
from jax.experimental.pallas import tpu_sc as plsc

VOCAB = 1000
B, T = 1024, 50
NTOK = B * T
NC, NS = 2, 16
NW = NC * NS
TPW = NTOK // NW
CHUNK = 32
NCH = TPW // CHUNK

_mesh = plsc.VectorSubcoreMesh(core_axis_name="c", subcore_axis_name="s")


def _lse_body(t_ref, out_ref):
    t = t_ref[...]
    m = jnp.max(t, axis=1)
    s = jnp.sum(jnp.exp(t - m[:, None]), axis=1)
    out_ref[0, 0, :] = m + jnp.log(s)


def _row_lse(table):
    out = pl.pallas_call(
        _lse_body,
        grid=(5,),
        in_specs=[pl.BlockSpec((200, VOCAB), lambda i: (i, 0))],
        out_specs=pl.BlockSpec((1, 1, 200), lambda i: (i, 0, 0)),
        out_shape=jax.ShapeDtypeStruct((5, 1, 200), jnp.float32),
    )(table)
    return jnp.pad(out.reshape(VOCAB), (0, 1024 - VOCAB))


@functools.partial(
    pl.kernel,
    mesh=_mesh,
    compiler_params=pltpu.CompilerParams(
        needs_layout_passes=False, use_tc_tiling_on_sc=False),
    out_type=[
        jax.ShapeDtypeStruct((NTOK, VOCAB), jnp.float32),
        jax.ShapeDtypeStruct((NW, 16), jnp.float32),
    ],
    scratch_types=[
        pltpu.VMEM((NCH, CHUNK), jnp.int32),
        pltpu.VMEM((NCH, CHUNK), jnp.int32),
        pltpu.VMEM((1024,), jnp.float32),
        pltpu.VMEM((CHUNK, VOCAB), jnp.float32),
        pltpu.VMEM((16,), jnp.float32),
        pltpu.SemaphoreType.DMA,
    ],
)
def _gather_loss(table_hbm, idx_hbm, tgt_hbm, lse_hbm, out_hbm, part_hbm,
                 idx_v, tgt_v, lse_v, rows_v, acc_v, sem):
    wid = lax.axis_index("s") * NC + lax.axis_index("c")
    base = wid * TPW
    pltpu.sync_copy(idx_hbm.at[wid], idx_v)
    pltpu.sync_copy(tgt_hbm.at[wid], tgt_v)
    pltpu.sync_copy(lse_hbm, lse_v)

    def body(c, acc):
        pltpu.async_copy(table_hbm.at[idx_v.at[c]], rows_v, sem).wait()
        pltpu.sync_copy(rows_v, out_hbm.at[pl.ds(base + c * CHUNK, CHUNK)])
        for j in range(CHUNK // 16):
            ids = idx_v[c, pl.ds(j * 16, 16)]
            tgs = tgt_v[c, pl.ds(j * 16, 16)]
            rloc = lax.iota(jnp.int32, 16) + j * 16
            lse16 = plsc.load_gather(lse_v, [ids])
            e16 = plsc.load_gather(rows_v, [rloc, tgs])
            acc = acc + lse16 - e16
        return acc

    acc = lax.fori_loop(0, NCH, body, jnp.zeros((16,), jnp.float32))
    acc_v[...] = acc
    pltpu.sync_copy(acc_v, part_hbm.at[wid])


def _combine_body(p_ref, o_ref):
    o_ref[...] = (jnp.sum(p_ref[...]) / NTOK).reshape(1, 1)


def _combine(parts):
    out = pl.pallas_call(
        _combine_body,
        out_shape=jax.ShapeDtypeStruct((1, 1), jnp.float32),
    )(parts)
    return out[0, 0]


def kernel(idx, targets, table):
    idx3 = idx.reshape(-1).astype(jnp.int32).reshape(NW, NCH, CHUNK)
    tgt3 = targets.reshape(-1).astype(jnp.int32).reshape(NW, NCH, CHUNK)
    lse = _row_lse(table)
    flat, parts = _gather_loss(table, idx3, tgt3, lse)
    return flat.reshape(B, T, VOCAB), _combine(parts)


# speedup vs baseline: 1.4183x; 1.0386x over previous
import functools

import jax
import jax.numpy as jnp
from jax import lax
from jax.experimental import pallas as pl
from jax.experimental.pallas import tpu as pltpu
from jax.experimental.pallas import tpu_sc as plsc

VOCAB = 1000
B, T = 1024, 50
NTOK = B * T
NC, NS = 2, 16
NW = NC * NS
TPW = NTOK // NW
CHUNK = 32
NCH = TPW // CHUNK

_mesh = plsc.VectorSubcoreMesh(core_axis_name="c", subcore_axis_name="s")


def _lse_body(t_ref, out_ref):
    t = t_ref[...]
    m = jnp.max(t, axis=1)
    s = jnp.sum(jnp.exp(t - m[:, None]), axis=1)
    out_ref[0, 0, :] = m + jnp.log(s)


def _row_lse(table):
    out = pl.pallas_call(
        _lse_body,
        grid=(5,),
        in_specs=[pl.BlockSpec((200, VOCAB), lambda i: (i, 0))],
        out_specs=pl.BlockSpec((1, 1, 200), lambda i: (i, 0, 0)),
        out_shape=jax.ShapeDtypeStruct((5, 1, 200), jnp.float32),
    )(table)
    return jnp.pad(out.reshape(VOCAB), (0, 1024 - VOCAB))


@functools.partial(
    pl.kernel,
    mesh=_mesh,
    compiler_params=pltpu.CompilerParams(
        needs_layout_passes=False, use_tc_tiling_on_sc=False),
    out_type=[
        jax.ShapeDtypeStruct((NTOK, VOCAB), jnp.float32),
        jax.ShapeDtypeStruct((NW, 16), jnp.float32),
    ],
    scratch_types=[
        pltpu.VMEM((NCH, CHUNK), jnp.int32),
        pltpu.VMEM((NCH, CHUNK), jnp.int32),
        pltpu.VMEM((1024,), jnp.float32),
        pltpu.VMEM((CHUNK, VOCAB), jnp.float32),
        pltpu.VMEM((CHUNK, VOCAB), jnp.float32),
        pltpu.VMEM((16,), jnp.float32),
        pltpu.SemaphoreType.DMA,
        pltpu.SemaphoreType.DMA,
        pltpu.SemaphoreType.DMA,
        pltpu.SemaphoreType.DMA,
    ],
)
def _gather_loss(table_hbm, idx_hbm, tgt_hbm, lse_hbm, out_hbm, part_hbm,
                 idx_v, tgt_v, lse_v, bufa, bufb, acc_v, sga, sgb, swa, swb):
    wid = lax.axis_index("s") * NC + lax.axis_index("c")
    base = wid * TPW
    pltpu.sync_copy(idx_hbm.at[wid], idx_v)
    pltpu.sync_copy(tgt_hbm.at[wid], tgt_v)
    pltpu.sync_copy(lse_hbm, lse_v)

    def start_gather(c, buf, sem):
        pltpu.async_copy(table_hbm.at[idx_v.at[c]], buf, sem)

    def wait_gather(c, buf, sem):
        pltpu.make_async_copy(table_hbm.at[idx_v.at[c]], buf, sem).wait()

    def start_write(c, buf, sem):
        pltpu.async_copy(buf, out_hbm.at[pl.ds(base + c * CHUNK, CHUNK)], sem)

    def wait_write(c, buf, sem):
        pltpu.make_async_copy(
            buf, out_hbm.at[pl.ds(base + c * CHUNK, CHUNK)], sem).wait()

    def loss(c, buf, acc):
        for j in range(CHUNK // 16):
            ids = idx_v[c, pl.ds(j * 16, 16)]
            tgs = tgt_v[c, pl.ds(j * 16, 16)]
            rloc = lax.iota(jnp.int32, 16) + j * 16
            lse16 = plsc.load_gather(lse_v, [ids])
            e16 = plsc.load_gather(buf, [rloc, tgs])
            acc = acc + lse16 - e16
        return acc

    NP = NCH // 2
    start_gather(0, bufa, sga)

    def pair(p, acc):
        c0 = 2 * p
        c1 = c0 + 1
        wait_gather(c0, bufa, sga)

        @pl.when(p > 0)
        def _():
            wait_write(c1 - 2, bufb, swb)

        start_gather(c1, bufb, sgb)
        acc = loss(c0, bufa, acc)
        start_write(c0, bufa, swa)
        wait_gather(c1, bufb, sgb)

        @pl.when(p < NP - 1)
        def _():
            wait_write(c0, bufa, swa)
            start_gather(c0 + 2, bufa, sga)

        acc = loss(c1, bufb, acc)
        start_write(c1, bufb, swb)
        return acc

    acc = lax.fori_loop(0, NP, pair, jnp.zeros((16,), jnp.float32))
    wait_write(NCH - 2, bufa, swa)
    wait_write(NCH - 1, bufb, swb)
    acc_v[...] = acc
    pltpu.sync_copy(acc_v, part_hbm.at[wid])


def _combine_body(p_ref, o_ref):
    o_ref[...] = (jnp.sum(p_ref[...]) / NTOK).reshape(1, 1)


def _combine(parts):
    out = pl.pallas_call(
        _combine_body,
        out_shape=jax.ShapeDtypeStruct((1, 1), jnp.float32),
    )(parts)
    return out[0, 0]


def kernel(idx, targets, table):
    idx3 = idx.reshape(-1).astype(jnp.int32).reshape(NW, NCH, CHUNK)
    tgt3 = targets.reshape(-1).astype(jnp.int32).reshape(NW, NCH, CHUNK)
    lse = _row_lse(table)
    flat, parts = _gather_loss(table, idx3, tgt3, lse)
    return flat.reshape(B, T, VOCAB), _combine(parts)


# speedup vs baseline: 1.4225x; 1.0030x over previous
import functools

import jax
import jax.numpy as jnp
from jax import lax
from jax.experimental import pallas as pl
from jax.experimental.pallas import tpu as pltpu
from jax.experimental.pallas import tpu_sc as plsc

VOCAB = 1000
B, T = 1024, 50
NTOK = B * T
NC, NS = 2, 16
NW = NC * NS
TPW = NTOK // NW
CHUNK = T
NCH = TPW // CHUNK

_mesh = plsc.VectorSubcoreMesh(core_axis_name="c", subcore_axis_name="s")


def _lse_body(t_ref, out_ref):
    t = t_ref[...]
    m = jnp.max(t, axis=1)
    s = jnp.sum(jnp.exp(t - m[:, None]), axis=1)
    out_ref[0, 0, :] = m + jnp.log(s)


def _row_lse(table):
    out = pl.pallas_call(
        _lse_body,
        grid=(5,),
        in_specs=[pl.BlockSpec((200, VOCAB), lambda i: (i, 0))],
        out_specs=pl.BlockSpec((1, 1, 200), lambda i: (i, 0, 0)),
        out_shape=jax.ShapeDtypeStruct((5, 1, 200), jnp.float32),
    )(table)
    return jnp.pad(out.reshape(VOCAB), (0, 1024 - VOCAB))


@functools.partial(
    pl.kernel,
    mesh=_mesh,
    compiler_params=pltpu.CompilerParams(
        needs_layout_passes=False, use_tc_tiling_on_sc=False),
    out_type=[
        jax.ShapeDtypeStruct((B, T, VOCAB), jnp.float32),
        jax.ShapeDtypeStruct((NW, 16), jnp.float32),
    ],
    scratch_types=[
        pltpu.VMEM((NCH, CHUNK), jnp.int32),
        pltpu.VMEM((NCH, CHUNK), jnp.int32),
        pltpu.VMEM((1024,), jnp.float32),
        pltpu.VMEM((CHUNK, VOCAB), jnp.float32),
        pltpu.VMEM((CHUNK, VOCAB), jnp.float32),
        pltpu.VMEM((16,), jnp.float32),
        pltpu.SemaphoreType.DMA,
        pltpu.SemaphoreType.DMA,
        pltpu.SemaphoreType.DMA,
        pltpu.SemaphoreType.DMA,
    ],
)
def _gather_loss(table_hbm, idx_hbm, tgt_hbm, lse_hbm, out_hbm, part_hbm,
                 idx_v, tgt_v, lse_v, bufa, bufb, acc_v, sga, sgb, swa, swb):
    wid = lax.axis_index("s") * NC + lax.axis_index("c")
    bbase = wid * NCH
    pltpu.sync_copy(idx_hbm.at[wid], idx_v)
    pltpu.sync_copy(tgt_hbm.at[wid], tgt_v)
    pltpu.sync_copy(lse_hbm, lse_v)

    def start_gather(c, buf, sem):
        pltpu.async_copy(table_hbm.at[idx_v.at[c]], buf, sem)

    def wait_gather(c, buf, sem):
        pltpu.make_async_copy(table_hbm.at[idx_v.at[c]], buf, sem).wait()

    def start_write(c, buf, sem):
        pltpu.async_copy(buf, out_hbm.at[bbase + c], sem)

    def wait_write(c, buf, sem):
        pltpu.make_async_copy(buf, out_hbm.at[bbase + c], sem).wait()

    def loss(c, buf, acc):
        for j in range(4):
            off = 34 if j == 3 else j * 16
            ids = idx_v[c, pl.ds(off, 16)]
            tgs = tgt_v[c, pl.ds(off, 16)]
            rloc = lax.iota(jnp.int32, 16) + off
            lse16 = plsc.load_gather(lse_v, [ids])
            e16 = plsc.load_gather(buf, [rloc, tgs])
            d = lse16 - e16
            if j == 3:
                d = jnp.where(lax.iota(jnp.int32, 16) >= 14, d, 0.0)
            acc = acc + d
        return acc

    NP = NCH // 2
    start_gather(0, bufa, sga)

    def pair(p, acc):
        c0 = 2 * p
        c1 = c0 + 1
        wait_gather(c0, bufa, sga)

        @pl.when(p > 0)
        def _():
            wait_write(c1 - 2, bufb, swb)

        start_gather(c1, bufb, sgb)
        acc = loss(c0, bufa, acc)
        start_write(c0, bufa, swa)
        wait_gather(c1, bufb, sgb)

        @pl.when(p < NP - 1)
        def _():
            wait_write(c0, bufa, swa)
            start_gather(c0 + 2, bufa, sga)

        acc = loss(c1, bufb, acc)
        start_write(c1, bufb, swb)
        return acc

    acc = lax.fori_loop(0, NP, pair, jnp.zeros((16,), jnp.float32))
    wait_write(NCH - 2, bufa, swa)
    wait_write(NCH - 1, bufb, swb)
    acc_v[...] = acc
    pltpu.sync_copy(acc_v, part_hbm.at[wid])


def _combine_body(p_ref, o_ref):
    o_ref[...] = (jnp.sum(p_ref[...]) / NTOK).reshape(1, 1)


def _combine(parts):
    out = pl.pallas_call(
        _combine_body,
        out_shape=jax.ShapeDtypeStruct((1, 1), jnp.float32),
    )(parts)
    return out[0, 0]


def kernel(idx, targets, table):
    idx3 = idx.reshape(-1).astype(jnp.int32).reshape(NW, NCH, CHUNK)
    tgt3 = targets.reshape(-1).astype(jnp.int32).reshape(NW, NCH, CHUNK)
    lse = _row_lse(table)
    logits, parts = _gather_loss(table, idx3, tgt3, lse)
    return logits, _combine(parts)
